# baseline (speedup 1.0000x reference)
import functools

import jax
import jax.numpy as jnp
from jax import lax
from jax.experimental import pallas as pl
from jax.experimental.pallas import tpu as pltpu
from jax.experimental.pallas import tpu_sc as plsc

_N = 10000
_E = 320000
_H = 128
_NCONV = 4
_NGRAPH = 256
_NLEV = 50

_EP = 327680
_ER = _EP // _H
_NC = 2
_NS = 16
_NW = _NC * _NS
_EPW = _EP // _NW
_CH = 80
_NCHUNK = _EPW // _CH
_NPAD = 10240
_RPS = _NPAD // _NS
_RZ = 128
_LANES = _H // 16

_sc_mesh = plsc.VectorSubcoreMesh(
    core_axis_name="c", subcore_axis_name="s", num_cores=_NC, num_subcores=_NS)


@functools.partial(
    pl.kernel,
    out_type=jax.ShapeDtypeStruct((_NC * _NPAD, _H), jnp.float32),
    mesh=_sc_mesh,
    scratch_types=[
        [pltpu.VMEM((_CH,), jnp.int32)] * 2,
        [pltpu.VMEM((_CH,), jnp.int32)] * 2,
        [pltpu.VMEM((_CH, _H), jnp.float32)] * 2,
        [pltpu.VMEM((_CH, _H), jnp.float32)] * 2,
        pltpu.VMEM_SHARED((_NPAD, _H), jnp.float32),
        [pltpu.SemaphoreType.DMA] * 2,
        [pltpu.SemaphoreType.DMA] * 2,
    ],
)
def _gin_aggregate(x_hbm, bond_hbm, src_hbm, dst_hbm, out_hbm,
                   si, di, rows, bondb, agg_sh, gsem, bsem):
    cid = lax.axis_index("c")
    sid = lax.axis_index("s")

    def _zero_row(i, _):
        for j in range(_LANES):
            rows[0][i, pl.ds(j * 16, 16)] = jnp.zeros((16,), jnp.float32)
        return 0
    lax.fori_loop(0, _CH, _zero_row, 0)
    nbase = sid * _RPS
    for k in range(_RPS // _CH):
        pltpu.sync_copy(rows[0], agg_sh.at[pl.ds(nbase + k * _CH, _CH)])
    plsc.subcore_barrier()

    ebase = (cid * _NS + sid) * _EPW

    def _issue(c, b):
        off = ebase + c * _CH
        pltpu.sync_copy(src_hbm.at[pl.ds(off, _CH)], si[b])
        pltpu.sync_copy(dst_hbm.at[pl.ds(off, _CH)], di[b])
        pltpu.async_copy(bond_hbm.at[pl.ds(off, _CH)], bondb[b], bsem[b])
        pltpu.async_copy(x_hbm.at[si[b]], rows[b], gsem[b])

    def _finish(b):
        pltpu.make_async_copy(
            bond_hbm.at[pl.ds(0, _CH)], bondb[b], bsem[b]).wait()
        pltpu.make_async_copy(x_hbm.at[si[b]], rows[b], gsem[b]).wait()

        @plsc.parallel_loop(0, _CH, unroll=4)
        def _row(r):
            for j in range(_LANES):
                sl = pl.ds(j * 16, 16)
                rows[b][r, sl] = jnp.maximum(
                    rows[b][r, sl] + bondb[b][r, sl], 0.0)
        pltpu.sync_copy(rows[b], agg_sh.at[di[b]], add=True)

    _issue(0, 0)

    def _pair(g, _):
        c0 = 2 * g
        _issue(c0 + 1, 1)
        _finish(0)
        _issue(c0 + 2, 0)
        _finish(1)
        return 0
    lax.fori_loop(0, _NCHUNK // 2 - 1, _pair, 0)
    _issue(_NCHUNK - 1, 1)
    _finish(0)
    _finish(1)
    plsc.subcore_barrier()

    obase = cid * _NPAD + nbase
    for k in range(_RPS // _CH):
        pltpu.sync_copy(agg_sh.at[pl.ds(nbase + k * _CH, _CH)], rows[0])
        pltpu.sync_copy(rows[0], out_hbm.at[pl.ds(obase + k * _CH, _CH)])


@functools.partial(
    pl.kernel,
    out_type=jax.ShapeDtypeStruct((_EP, _H), jnp.float32),
    mesh=_sc_mesh,
    scratch_types=[
        [pltpu.VMEM((_CH,), jnp.int32)] * 2,
        [pltpu.VMEM((_CH,), jnp.int32)] * 2,
        [pltpu.VMEM((_CH, _H), jnp.float32)] * 2,
        [pltpu.VMEM((_CH, _H), jnp.float32)] * 2,
        [pltpu.SemaphoreType.DMA] * 2,
        [pltpu.SemaphoreType.DMA] * 2,
    ],
)
def _edge_pair_product(x_hbm, src_hbm, dst_hbm, out_hbm,
                       si, di, av, bv, sem1, sem2):
    cid = lax.axis_index("c")
    sid = lax.axis_index("s")
    ebase = (cid * _NS + sid) * _EPW

    def _issue(c, b):
        off = ebase + c * _CH
        pltpu.sync_copy(src_hbm.at[pl.ds(off, _CH)], si[b])
        pltpu.sync_copy(dst_hbm.at[pl.ds(off, _CH)], di[b])
        pltpu.async_copy(x_hbm.at[si[b]], av[b], sem1[b])
        pltpu.async_copy(x_hbm.at[di[b]], bv[b], sem2[b])

    def _finish(c, b):
        off = ebase + c * _CH
        pltpu.make_async_copy(x_hbm.at[si[b]], av[b], sem1[b]).wait()
        pltpu.make_async_copy(x_hbm.at[di[b]], bv[b], sem2[b]).wait()

        @plsc.parallel_loop(0, _CH, unroll=4)
        def _row(r):
            for j in range(_LANES):
                sl = pl.ds(j * 16, 16)
                av[b][r, sl] = av[b][r, sl] * bv[b][r, sl]
        pltpu.sync_copy(av[b], out_hbm.at[pl.ds(off, _CH)])

    _issue(0, 0)

    def _pair(g, _):
        c0 = 2 * g
        _issue(c0 + 1, 1)
        _finish(c0, 0)
        _issue(c0 + 2, 0)
        _finish(c0 + 1, 1)
        return 0
    lax.fori_loop(0, _NCHUNK // 2 - 1, _pair, 0)
    _issue(_NCHUNK - 1, 1)
    _finish(_NCHUNK - 2, 0)
    _finish(_NCHUNK - 1, 1)


@functools.partial(
    pl.kernel,
    out_type=(jax.ShapeDtypeStruct((_EP,), jnp.float32),
              jax.ShapeDtypeStruct((_EP,), jnp.float32)),
    mesh=_sc_mesh,
    scratch_types=[
        pltpu.VMEM((_N,), jnp.int32),
        pltpu.VMEM((64,), jnp.float32),
        pltpu.VMEM((_NGRAPH,), jnp.int32),
        pltpu.VMEM((_NGRAPH,), jnp.float32),
        pltpu.VMEM((_NGRAPH,), jnp.float32),
        pltpu.VMEM((_EPW,), jnp.int32),
        pltpu.VMEM((_EPW,), jnp.float32),
        pltpu.VMEM((_EPW,), jnp.float32),
        pltpu.VMEM((_EPW,), jnp.float32),
    ],
    compiler_params=pltpu.CompilerParams(needs_layout_passes=False),
)
def _edge_sigma_target(batch_hbm, sig_hbm, nl_hbm, src_hbm, dn_hbm,
                       sig_out_hbm, tgt_out_hbm,
                       bt_v, sg_v, nl_v, us_v, ni_v, src_v, dn_v, so_v, to_v):
    cid = lax.axis_index("c")
    sid = lax.axis_index("s")
    pltpu.sync_copy(batch_hbm, bt_v)
    pltpu.sync_copy(sig_hbm, sg_v)
    pltpu.sync_copy(nl_hbm, nl_v)

    def _us(i, _):
        sl = pl.ds(i * 16, 16)
        sg = plsc.load_gather(sg_v, [nl_v[sl]])
        us_v[sl] = sg
        ni_v[sl] = -1.0 / (sg * sg)
        return 0
    lax.fori_loop(0, _NGRAPH // 16, _us, 0)

    ebase = (cid * _NS + sid) * _EPW
    pltpu.sync_copy(src_hbm.at[pl.ds(ebase, _EPW)], src_v)
    pltpu.sync_copy(dn_hbm.at[pl.ds(ebase, _EPW)], dn_v)

    @plsc.parallel_loop(0, _EPW // 16, unroll=4)
    def _e(i):
        sl = pl.ds(i * 16, 16)
        g = plsc.load_gather(bt_v, [src_v[sl]])
        so_v[sl] = plsc.load_gather(us_v, [g])
        to_v[sl] = plsc.load_gather(ni_v, [g]) * dn_v[sl]
    pltpu.sync_copy(so_v, sig_out_hbm.at[pl.ds(ebase, _EPW)])
    pltpu.sync_copy(to_v, tgt_out_hbm.at[pl.ds(ebase, _EPW)])


_BER = 32
_BE = _BER * _H
_BN = 2000


def _bond_tc(pd2, et2, Wi1, bi1, Wi2, bi2, emb_pad):
    def body(pd_ref, et_ref, w1_ref, b1_ref, w2_ref, b2_ref, emb_ref,
             out_ref):
        pdT = jnp.transpose(pd_ref[...])
        etT = jnp.transpose(et_ref[...])
        iota = lax.broadcasted_iota(jnp.int32, (_H, _H), 1).astype(jnp.float32)
        a_cols = []
        oh_cols = []
        for r in range(_BER):
            pcol = pdT[:, r:r + 1]
            a_cols.append(jnp.maximum(pcol * w1_ref[...] + b1_ref[...], 0.0))
            oh_cols.append((etT[:, r:r + 1] == iota).astype(jnp.float32))
        a = jnp.concatenate(a_cols, 0)
        oh = jnp.concatenate(oh_cols, 0)
        demb = jnp.dot(a, w2_ref[...],
                       preferred_element_type=jnp.float32) + b2_ref[...]
        battr = jnp.dot(oh, emb_ref[...],
                        preferred_element_type=jnp.float32,
                        precision=lax.Precision.HIGHEST)
        out_ref[...] = demb * battr

    return pl.pallas_call(
        body,
        grid=(_EP // _BE,),
        in_specs=[
            pl.BlockSpec((_BER, _H), lambda i: (i, 0)),
            pl.BlockSpec((_BER, _H), lambda i: (i, 0)),
            pl.BlockSpec((1, _H), lambda i: (0, 0)),
            pl.BlockSpec((1, _H), lambda i: (0, 0)),
            pl.BlockSpec((_H, _H), lambda i: (0, 0)),
            pl.BlockSpec((1, _H), lambda i: (0, 0)),
            pl.BlockSpec((_H, _H), lambda i: (0, 0)),
        ],
        out_specs=pl.BlockSpec((_BE, _H), lambda i: (i, 0)),
        out_shape=jax.ShapeDtypeStruct((_EP, _H), jnp.float32),
    )(pd2, et2, Wi1, bi1, Wi2, bi2, emb_pad)


def _node_emb_tc(nt, emb_pad):
    def body(nt_ref, emb_ref, out_ref):
        onehot = (nt_ref[...] == lax.broadcasted_iota(
            jnp.int32, (_BN, _H), 1)).astype(jnp.float32)
        out_ref[...] = jnp.dot(onehot, emb_ref[...],
                               preferred_element_type=jnp.float32,
                               precision=lax.Precision.HIGHEST)

    return pl.pallas_call(
        body,
        grid=(_N // _BN,),
        in_specs=[
            pl.BlockSpec((_BN, 1), lambda i: (i, 0)),
            pl.BlockSpec((_H, _H), lambda i: (0, 0)),
        ],
        out_specs=pl.BlockSpec((_BN, _H), lambda i: (i, 0)),
        out_shape=jax.ShapeDtypeStruct((_N, _H), jnp.float32),
    )(nt, emb_pad)


def _node_mlp_tc(x, p0, p1, W1, b1, W2, b2):
    def body(x_ref, p0_ref, p1_ref, w1_ref, b1_ref, w2_ref, b2_ref, out_ref):
        h = x_ref[...] + p0_ref[...] + p1_ref[...]
        t = jnp.maximum(jnp.dot(h, w1_ref[...],
                                preferred_element_type=jnp.float32)
                        + b1_ref[...], 0.0)
        u = jnp.dot(t, w2_ref[...],
                    preferred_element_type=jnp.float32) + b2_ref[...]
        out_ref[...] = jnp.maximum(u, 0.0) + x_ref[...]

    return pl.pallas_call(
        body,
        grid=(_N // _BN,),
        in_specs=[
            pl.BlockSpec((_BN, _H), lambda i: (i, 0)),
            pl.BlockSpec((_BN, _H), lambda i: (i, 0)),
            pl.BlockSpec((_BN, _H), lambda i: (i, 0)),
            pl.BlockSpec((_H, _H), lambda i: (0, 0)),
            pl.BlockSpec((1, _H), lambda i: (0, 0)),
            pl.BlockSpec((_H, _H), lambda i: (0, 0)),
            pl.BlockSpec((1, _H), lambda i: (0, 0)),
        ],
        out_specs=pl.BlockSpec((_BN, _H), lambda i: (i, 0)),
        out_shape=jax.ShapeDtypeStruct((_N, _H), jnp.float32),
    )(x, p0, p1, W1, b1, W2, b2)


def _score_tc(prod, bond, sig2, Wo1a, Wo1b, bo1, Wo2p, bo2p, Wo3p, bo3):
    def body(p_ref, b_ref, s_ref, w1a_ref, w1b_ref, b1_ref, w2_ref, b2_ref,
             w3_ref, b3_ref, out_ref):
        s1 = jnp.maximum(
            jnp.dot(p_ref[...], w1a_ref[...],
                    preferred_element_type=jnp.float32)
            + jnp.dot(b_ref[...], w1b_ref[...],
                      preferred_element_type=jnp.float32)
            + b1_ref[...], 0.0)
        s2 = jnp.maximum(jnp.dot(s1, w2_ref[...],
                                 preferred_element_type=jnp.float32)
                         + b2_ref[...], 0.0)
        raw = jnp.dot(s2, w3_ref[...],
                      preferred_element_type=jnp.float32) + b3_ref[...]
        sgT = jnp.transpose(s_ref[...])
        cols = []
        for r in range(_BER):
            rcol = raw[r * _H:(r + 1) * _H, 0:1]
            cols.append(rcol * (1.0 / sgT[:, r:r + 1]))
        outT = jnp.concatenate(cols, 1)
        out_ref[...] = jnp.transpose(outT)

    return pl.pallas_call(
        body,
        grid=(_EP // _BE,),
        in_specs=[
            pl.BlockSpec((_BE, _H), lambda i: (i, 0)),
            pl.BlockSpec((_BE, _H), lambda i: (i, 0)),
            pl.BlockSpec((_BER, _H), lambda i: (i, 0)),
            pl.BlockSpec((_H, _H), lambda i: (0, 0)),
            pl.BlockSpec((_H, _H), lambda i: (0, 0)),
            pl.BlockSpec((1, _H), lambda i: (0, 0)),
            pl.BlockSpec((_H, _H), lambda i: (0, 0)),
            pl.BlockSpec((1, _H), lambda i: (0, 0)),
            pl.BlockSpec((_H, 1), lambda i: (0, 0)),
            pl.BlockSpec((1, 1), lambda i: (0, 0)),
        ],
        out_specs=pl.BlockSpec((_BER, _H), lambda i: (i, 0)),
        out_shape=jax.ShapeDtypeStruct((_ER, _H), jnp.float32),
    )(prod, bond, sig2, Wo1a, Wo1b, bo1, Wo2p, bo2p, Wo3p, bo3)


def kernel(node_type, edge_type, edge_index, batch, edge_length,
           node_emb, edge_emb, Wi1, bi1, Wi2, bi2,
           convW1, convb1, convW2, convb2,
           Wo1, bo1, Wo2, bo2, Wo3, bo3):
    f32 = jnp.float32
    i32 = jnp.int32
    sigmas = jnp.exp(
        jnp.linspace(jnp.log(10.0), jnp.log(0.01), _NLEV)).astype(f32)
    kn = jax.random.key(42)
    noise_level = jax.random.randint(
        jax.random.fold_in(kn, 0), (_NGRAPH,), 0, _NLEV)
    d_noise = jax.random.normal(
        jax.random.fold_in(kn, 1), (_E,), dtype=f32)

    npad = _EP - _E
    src = jnp.pad(edge_index[0].astype(i32), (0, npad))
    dst = edge_index[1].astype(i32)
    dst_gin = jnp.pad(dst, (0, npad), constant_values=_N + 8)
    dst_pair = jnp.pad(dst, (0, npad))
    sig_pad = jnp.pad(sigmas, (0, 64 - _NLEV))
    node_emb_pad = jnp.pad(node_emb, ((0, _H - node_emb.shape[0]), (0, 0)))
    edge_emb_pad = jnp.pad(edge_emb, ((0, _H - edge_emb.shape[0]), (0, 0)))

    pd2 = jnp.pad(edge_length[:, 0] + d_noise, (0, npad)).reshape(_ER, _H)
    et2 = jnp.pad(edge_type.astype(f32), (0, npad)).reshape(_ER, _H)
    dn_pad = jnp.pad(d_noise, (0, npad))

    bond = _bond_tc(pd2, et2, Wi1, bi1[None], Wi2, bi2[None], edge_emb_pad)
    x = _node_emb_tc(node_type.astype(i32)[:, None], node_emb_pad)
    for i in range(_NCONV):
        part = _gin_aggregate(x, bond, src, dst_gin)
        x = _node_mlp_tc(x, part[:_N], part[_NPAD:_NPAD + _N],
                         convW1[i], convb1[i][None],
                         convW2[i], convb2[i][None])

    prod = _edge_pair_product(x, src, dst_pair)
    sig_flat, tgt_flat = _edge_sigma_target(
        batch.astype(i32), sig_pad, noise_level.astype(i32), src, dn_pad)

    Wo2p = jnp.pad(Wo2, ((0, 0), (0, _H - Wo2.shape[1])))
    bo2p = jnp.pad(bo2, (0, _H - bo2.shape[0]))
    Wo3p = jnp.pad(Wo3, ((0, _H - Wo3.shape[0]), (0, 0)))
    scores2 = _score_tc(prod, bond, sig_flat.reshape(_ER, _H),
                        Wo1[:_H], Wo1[_H:], bo1[None],
                        Wo2p, bo2p[None], Wo3p, bo3[None])
    scores = scores2.reshape(_EP)[:_E, None]
    return (scores, tgt_flat[:_E, None], sig_flat[:_E, None])


# speedup vs baseline: 1.8431x; 1.8431x over previous
import functools

import jax
import jax.numpy as jnp
from jax import lax
from jax.experimental import pallas as pl
from jax.experimental.pallas import tpu as pltpu
from jax.experimental.pallas import tpu_sc as plsc

_N = 10000
_E = 320000
_H = 128
_NCONV = 4
_NGRAPH = 256
_NLEV = 50

_EP = 327680
_ER = _EP // _H
_NC = 2
_NS = 16
_NW = _NC * _NS
_EPW = _EP // _NW
_CH = 80
_NCHUNK = _EPW // _CH
_NPAD = 10240
_RPS = _NPAD // _NS
_RZ = 128
_LANES = _H // 16

_sc_mesh = plsc.VectorSubcoreMesh(
    core_axis_name="c", subcore_axis_name="s", num_cores=_NC, num_subcores=_NS)


@functools.partial(
    pl.kernel,
    out_type=jax.ShapeDtypeStruct((_NC * _NPAD, _H), jnp.float32),
    mesh=_sc_mesh,
    scratch_types=[
        [pltpu.VMEM((_CH,), jnp.int32)] * 2,
        [pltpu.VMEM((_CH,), jnp.int32)] * 2,
        [pltpu.VMEM((_CH, _H), jnp.float32)] * 2,
        [pltpu.VMEM((_CH, _H), jnp.float32)] * 2,
        pltpu.VMEM_SHARED((_NPAD, _H), jnp.float32),
        [pltpu.SemaphoreType.DMA] * 2,
        [pltpu.SemaphoreType.DMA] * 2,
    ],
)
def _gin_aggregate(x_hbm, bond_hbm, src_hbm, dst_hbm, out_hbm,
                   si, di, rows, bondb, agg_sh, gsem, bsem):
    cid = lax.axis_index("c")
    sid = lax.axis_index("s")

    def _zero_row(i, _):
        for j in range(_LANES):
            rows[0][i, pl.ds(j * 16, 16)] = jnp.zeros((16,), jnp.float32)
        return 0
    lax.fori_loop(0, _CH, _zero_row, 0)
    nbase = sid * _RPS
    for k in range(_RPS // _CH):
        pltpu.sync_copy(rows[0], agg_sh.at[pl.ds(nbase + k * _CH, _CH)])
    plsc.subcore_barrier()

    ebase = (cid * _NS + sid) * _EPW

    def _issue(c, b):
        off = ebase + c * _CH
        pltpu.sync_copy(src_hbm.at[pl.ds(off, _CH)], si[b])
        pltpu.sync_copy(dst_hbm.at[pl.ds(off, _CH)], di[b])
        pltpu.async_copy(bond_hbm.at[pl.ds(off, _CH)], bondb[b], bsem[b])
        pltpu.async_copy(x_hbm.at[si[b]], rows[b], gsem[b])

    def _finish(b):
        pltpu.make_async_copy(
            bond_hbm.at[pl.ds(0, _CH)], bondb[b], bsem[b]).wait()
        pltpu.make_async_copy(x_hbm.at[si[b]], rows[b], gsem[b]).wait()

        @plsc.parallel_loop(0, _CH, unroll=4)
        def _row(r):
            for j in range(_LANES):
                sl = pl.ds(j * 16, 16)
                rows[b][r, sl] = jnp.maximum(
                    rows[b][r, sl] + bondb[b][r, sl], 0.0)
        pltpu.sync_copy(rows[b], agg_sh.at[di[b]], add=True)

    _issue(0, 0)

    def _pair(g, _):
        c0 = 2 * g
        _issue(c0 + 1, 1)
        _finish(0)
        _issue(c0 + 2, 0)
        _finish(1)
        return 0
    lax.fori_loop(0, _NCHUNK // 2 - 1, _pair, 0)
    _issue(_NCHUNK - 1, 1)
    _finish(0)
    _finish(1)
    plsc.subcore_barrier()

    obase = cid * _NPAD + nbase
    for k in range(_RPS // _CH):
        pltpu.sync_copy(agg_sh.at[pl.ds(nbase + k * _CH, _CH)], rows[0])
        pltpu.sync_copy(rows[0], out_hbm.at[pl.ds(obase + k * _CH, _CH)])


@functools.partial(
    pl.kernel,
    out_type=jax.ShapeDtypeStruct((_EP, _H), jnp.float32),
    mesh=_sc_mesh,
    scratch_types=[
        [pltpu.VMEM((_CH,), jnp.int32)] * 2,
        [pltpu.VMEM((_CH,), jnp.int32)] * 2,
        [pltpu.VMEM((_CH, _H), jnp.float32)] * 2,
        [pltpu.VMEM((_CH, _H), jnp.float32)] * 2,
        [pltpu.SemaphoreType.DMA] * 2,
        [pltpu.SemaphoreType.DMA] * 2,
    ],
)
def _edge_pair_product(x_hbm, src_hbm, dst_hbm, out_hbm,
                       si, di, av, bv, sem1, sem2):
    cid = lax.axis_index("c")
    sid = lax.axis_index("s")
    ebase = (cid * _NS + sid) * _EPW

    def _issue(c, b):
        off = ebase + c * _CH
        pltpu.sync_copy(src_hbm.at[pl.ds(off, _CH)], si[b])
        pltpu.sync_copy(dst_hbm.at[pl.ds(off, _CH)], di[b])
        pltpu.async_copy(x_hbm.at[si[b]], av[b], sem1[b])
        pltpu.async_copy(x_hbm.at[di[b]], bv[b], sem2[b])

    def _finish(c, b):
        off = ebase + c * _CH
        pltpu.make_async_copy(x_hbm.at[si[b]], av[b], sem1[b]).wait()
        pltpu.make_async_copy(x_hbm.at[di[b]], bv[b], sem2[b]).wait()

        @plsc.parallel_loop(0, _CH, unroll=4)
        def _row(r):
            for j in range(_LANES):
                sl = pl.ds(j * 16, 16)
                av[b][r, sl] = av[b][r, sl] * bv[b][r, sl]
        pltpu.sync_copy(av[b], out_hbm.at[pl.ds(off, _CH)])

    _issue(0, 0)

    def _pair(g, _):
        c0 = 2 * g
        _issue(c0 + 1, 1)
        _finish(c0, 0)
        _issue(c0 + 2, 0)
        _finish(c0 + 1, 1)
        return 0
    lax.fori_loop(0, _NCHUNK // 2 - 1, _pair, 0)
    _issue(_NCHUNK - 1, 1)
    _finish(_NCHUNK - 2, 0)
    _finish(_NCHUNK - 1, 1)


@functools.partial(
    pl.kernel,
    out_type=(jax.ShapeDtypeStruct((_EP,), jnp.float32),
              jax.ShapeDtypeStruct((_EP,), jnp.float32)),
    mesh=_sc_mesh,
    scratch_types=[
        pltpu.VMEM((_N,), jnp.int32),
        pltpu.VMEM((64,), jnp.float32),
        pltpu.VMEM((_NGRAPH,), jnp.int32),
        pltpu.VMEM((_NGRAPH,), jnp.float32),
        pltpu.VMEM((_NGRAPH,), jnp.float32),
        pltpu.VMEM((_EPW,), jnp.int32),
        pltpu.VMEM((_EPW,), jnp.float32),
        pltpu.VMEM((_EPW,), jnp.float32),
        pltpu.VMEM((_EPW,), jnp.float32),
    ],
    compiler_params=pltpu.CompilerParams(needs_layout_passes=False),
)
def _edge_sigma_target(batch_hbm, sig_hbm, nl_hbm, src_hbm, dn_hbm,
                       sig_out_hbm, tgt_out_hbm,
                       bt_v, sg_v, nl_v, us_v, ni_v, src_v, dn_v, so_v, to_v):
    cid = lax.axis_index("c")
    sid = lax.axis_index("s")
    pltpu.sync_copy(batch_hbm, bt_v)
    pltpu.sync_copy(sig_hbm, sg_v)
    pltpu.sync_copy(nl_hbm, nl_v)

    def _us(i, _):
        sl = pl.ds(i * 16, 16)
        sg = plsc.load_gather(sg_v, [nl_v[sl]])
        us_v[sl] = sg
        ni_v[sl] = -1.0 / (sg * sg)
        return 0
    lax.fori_loop(0, _NGRAPH // 16, _us, 0)

    ebase = (cid * _NS + sid) * _EPW
    pltpu.sync_copy(src_hbm.at[pl.ds(ebase, _EPW)], src_v)
    pltpu.sync_copy(dn_hbm.at[pl.ds(ebase, _EPW)], dn_v)

    @plsc.parallel_loop(0, _EPW // 16, unroll=4)
    def _e(i):
        sl = pl.ds(i * 16, 16)
        g = plsc.load_gather(bt_v, [src_v[sl]])
        so_v[sl] = plsc.load_gather(us_v, [g])
        to_v[sl] = plsc.load_gather(ni_v, [g]) * dn_v[sl]
    pltpu.sync_copy(so_v, sig_out_hbm.at[pl.ds(ebase, _EPW)])
    pltpu.sync_copy(to_v, tgt_out_hbm.at[pl.ds(ebase, _EPW)])


_BER = 32
_BE = _BER * _H
_BN = 2000


def _bond_tc(pd2, et2, Wi1, bi1, Wi2, bi2, emb_pad):
    def body(pd_ref, et_ref, w1_ref, b1_ref, w2_ref, b2_ref, emb_ref,
             out_ref):
        pdT = jnp.transpose(pd_ref[...])
        etT = jnp.transpose(et_ref[...])
        iota = lax.broadcasted_iota(jnp.int32, (_H, _H), 1).astype(jnp.float32)
        a_cols = []
        oh_cols = []
        for r in range(_BER):
            pcol = pdT[:, r:r + 1]
            a_cols.append(jnp.maximum(pcol * w1_ref[...] + b1_ref[...], 0.0))
            oh_cols.append((etT[:, r:r + 1] == iota).astype(jnp.float32))
        a = jnp.concatenate(a_cols, 0)
        oh = jnp.concatenate(oh_cols, 0)
        demb = jnp.dot(a, w2_ref[...],
                       preferred_element_type=jnp.float32) + b2_ref[...]
        battr = jnp.dot(oh, emb_ref[...],
                        preferred_element_type=jnp.float32,
                        precision=lax.Precision.HIGHEST)
        out_ref[...] = demb * battr

    return pl.pallas_call(
        body,
        grid=(_EP // _BE,),
        in_specs=[
            pl.BlockSpec((_BER, _H), lambda i: (i, 0)),
            pl.BlockSpec((_BER, _H), lambda i: (i, 0)),
            pl.BlockSpec((1, _H), lambda i: (0, 0)),
            pl.BlockSpec((1, _H), lambda i: (0, 0)),
            pl.BlockSpec((_H, _H), lambda i: (0, 0)),
            pl.BlockSpec((1, _H), lambda i: (0, 0)),
            pl.BlockSpec((_H, _H), lambda i: (0, 0)),
        ],
        out_specs=pl.BlockSpec((_BE, _H), lambda i: (i, 0)),
        out_shape=jax.ShapeDtypeStruct((_EP, _H), jnp.float32),
    )(pd2, et2, Wi1, bi1, Wi2, bi2, emb_pad)


def _node_emb_tc(nt, emb_pad):
    def body(nt_ref, emb_ref, out_ref):
        onehot = (nt_ref[...] == lax.broadcasted_iota(
            jnp.int32, (_BN, _H), 1)).astype(jnp.float32)
        out_ref[...] = jnp.dot(onehot, emb_ref[...],
                               preferred_element_type=jnp.float32,
                               precision=lax.Precision.HIGHEST)

    return pl.pallas_call(
        body,
        grid=(_N // _BN,),
        in_specs=[
            pl.BlockSpec((_BN, 1), lambda i: (i, 0)),
            pl.BlockSpec((_H, _H), lambda i: (0, 0)),
        ],
        out_specs=pl.BlockSpec((_BN, _H), lambda i: (i, 0)),
        out_shape=jax.ShapeDtypeStruct((_N, _H), jnp.float32),
    )(nt, emb_pad)


def _node_mlp_tc(x, p0, p1, W1, b1, W2, b2):
    def body(x_ref, p0_ref, p1_ref, w1_ref, b1_ref, w2_ref, b2_ref, out_ref):
        h = x_ref[...] + p0_ref[...] + p1_ref[...]
        t = jnp.maximum(jnp.dot(h, w1_ref[...],
                                preferred_element_type=jnp.float32)
                        + b1_ref[...], 0.0)
        u = jnp.dot(t, w2_ref[...],
                    preferred_element_type=jnp.float32) + b2_ref[...]
        out_ref[...] = jnp.maximum(u, 0.0) + x_ref[...]

    return pl.pallas_call(
        body,
        grid=(_N // _BN,),
        in_specs=[
            pl.BlockSpec((_BN, _H), lambda i: (i, 0)),
            pl.BlockSpec((_BN, _H), lambda i: (i, 0)),
            pl.BlockSpec((_BN, _H), lambda i: (i, 0)),
            pl.BlockSpec((_H, _H), lambda i: (0, 0)),
            pl.BlockSpec((1, _H), lambda i: (0, 0)),
            pl.BlockSpec((_H, _H), lambda i: (0, 0)),
            pl.BlockSpec((1, _H), lambda i: (0, 0)),
        ],
        out_specs=pl.BlockSpec((_BN, _H), lambda i: (i, 0)),
        out_shape=jax.ShapeDtypeStruct((_N, _H), jnp.float32),
    )(x, p0, p1, W1, b1, W2, b2)


def _score_tc(prod, bond, sig2, Wo1a, Wo1b, bo1, Wo2p, bo2p, Wo3p, bo3):
    def body(p_ref, b_ref, s_ref, w1a_ref, w1b_ref, b1_ref, w2_ref, b2_ref,
             w3_ref, b3_ref, out_ref):
        s1 = jnp.maximum(
            jnp.dot(p_ref[...], w1a_ref[...],
                    preferred_element_type=jnp.float32)
            + jnp.dot(b_ref[...], w1b_ref[...],
                      preferred_element_type=jnp.float32)
            + b1_ref[...], 0.0)
        s2 = jnp.maximum(jnp.dot(s1, w2_ref[...],
                                 preferred_element_type=jnp.float32)
                         + b2_ref[...], 0.0)
        raw = jnp.dot(s2, w3_ref[...],
                      preferred_element_type=jnp.float32) + b3_ref[...]
        sgT = jnp.transpose(s_ref[...])
        cols = []
        for r in range(_BER):
            rcol = raw[r * _H:(r + 1) * _H, 0:1]
            cols.append(rcol * (1.0 / sgT[:, r:r + 1]))
        outT = jnp.concatenate(cols, 1)
        out_ref[...] = jnp.transpose(outT)

    return pl.pallas_call(
        body,
        grid=(_EP // _BE,),
        in_specs=[
            pl.BlockSpec((_BE, _H), lambda i: (i, 0)),
            pl.BlockSpec((_BE, _H), lambda i: (i, 0)),
            pl.BlockSpec((_BER, _H), lambda i: (i, 0)),
            pl.BlockSpec((_H, _H), lambda i: (0, 0)),
            pl.BlockSpec((_H, _H), lambda i: (0, 0)),
            pl.BlockSpec((1, _H), lambda i: (0, 0)),
            pl.BlockSpec((_H, _H), lambda i: (0, 0)),
            pl.BlockSpec((1, _H), lambda i: (0, 0)),
            pl.BlockSpec((_H, 1), lambda i: (0, 0)),
            pl.BlockSpec((1, 1), lambda i: (0, 0)),
        ],
        out_specs=pl.BlockSpec((_BER, _H), lambda i: (i, 0)),
        out_shape=jax.ShapeDtypeStruct((_ER, _H), jnp.float32),
    )(prod, bond, sig2, Wo1a, Wo1b, bo1, Wo2p, bo2p, Wo3p, bo3)


def kernel(node_type, edge_type, edge_index, batch, edge_length,
           node_emb, edge_emb, Wi1, bi1, Wi2, bi2,
           convW1, convb1, convW2, convb2,
           Wo1, bo1, Wo2, bo2, Wo3, bo3):
    f32 = jnp.float32
    i32 = jnp.int32
    sigmas = jnp.exp(
        jnp.linspace(jnp.log(10.0), jnp.log(0.01), _NLEV)).astype(f32)
    kn = jax.random.key(42)
    noise_level = jax.random.randint(
        jax.random.fold_in(kn, 0), (_NGRAPH,), 0, _NLEV)
    d_noise = jax.random.normal(
        jax.random.fold_in(kn, 1), (_E,), dtype=f32)

    npad = _EP - _E
    fill = jnp.arange(npad, dtype=i32)
    src = jnp.concatenate([edge_index[0].astype(i32), fill % _N])
    dst = edge_index[1].astype(i32)
    dst_gin = jnp.concatenate([dst, _N + (fill % (_NPAD - _N))])
    dst_pair = jnp.concatenate([dst, fill % _N])
    sig_pad = jnp.pad(sigmas, (0, 64 - _NLEV))
    node_emb_pad = jnp.pad(node_emb, ((0, _H - node_emb.shape[0]), (0, 0)))
    edge_emb_pad = jnp.pad(edge_emb, ((0, _H - edge_emb.shape[0]), (0, 0)))

    pd2 = jnp.pad(edge_length[:, 0] + d_noise, (0, npad)).reshape(_ER, _H)
    et2 = jnp.pad(edge_type.astype(f32), (0, npad)).reshape(_ER, _H)
    dn_pad = jnp.pad(d_noise, (0, npad))

    bond = _bond_tc(pd2, et2, Wi1, bi1[None], Wi2, bi2[None], edge_emb_pad)
    x = _node_emb_tc(node_type.astype(i32)[:, None], node_emb_pad)
    for i in range(_NCONV):
        part = _gin_aggregate(x, bond, src, dst_gin)
        x = _node_mlp_tc(x, part[:_N], part[_NPAD:_NPAD + _N],
                         convW1[i], convb1[i][None],
                         convW2[i], convb2[i][None])

    prod = _edge_pair_product(x, src, dst_pair)
    sig_flat, tgt_flat = _edge_sigma_target(
        batch.astype(i32), sig_pad, noise_level.astype(i32), src, dn_pad)

    Wo2p = jnp.pad(Wo2, ((0, 0), (0, _H - Wo2.shape[1])))
    bo2p = jnp.pad(bo2, (0, _H - bo2.shape[0]))
    Wo3p = jnp.pad(Wo3, ((0, _H - Wo3.shape[0]), (0, 0)))
    scores2 = _score_tc(prod, bond, sig_flat.reshape(_ER, _H),
                        Wo1[:_H], Wo1[_H:], bo1[None],
                        Wo2p, bo2p[None], Wo3p, bo3[None])
    scores = scores2.reshape(_EP)[:_E, None]
    return (scores, tgt_flat[:_E, None], sig_flat[:_E, None])


# speedup vs baseline: 1.8773x; 1.0186x over previous
import functools

import jax
import jax.numpy as jnp
from jax import lax
from jax.experimental import pallas as pl
from jax.experimental.pallas import tpu as pltpu
from jax.experimental.pallas import tpu_sc as plsc

_N = 10000
_E = 320000
_H = 128
_NCONV = 4
_NGRAPH = 256
_NLEV = 50

_EP = 327680
_ER = _EP // _H
_NC = 2
_NS = 16
_NW = _NC * _NS
_EPW = _EP // _NW
_CH = 80
_NCHUNK = _EPW // _CH
_CHP = 128
_NCHUNKP = _EPW // _CHP
_NPAD = 10240
_RPS = _NPAD // _NS
_RZ = 128
_LANES = _H // 16

_sc_mesh = plsc.VectorSubcoreMesh(
    core_axis_name="c", subcore_axis_name="s", num_cores=_NC, num_subcores=_NS)


@functools.partial(
    pl.kernel,
    out_type=jax.ShapeDtypeStruct((_NC * _NPAD, _H), jnp.float32),
    mesh=_sc_mesh,
    scratch_types=[
        [pltpu.VMEM((_CH,), jnp.int32)] * 2,
        [pltpu.VMEM((_CH,), jnp.int32)] * 2,
        [pltpu.VMEM((_CH, _H), jnp.float32)] * 2,
        [pltpu.VMEM((_CH, _H), jnp.float32)] * 2,
        pltpu.VMEM_SHARED((_NPAD, _H), jnp.float32),
        [pltpu.SemaphoreType.DMA] * 2,
        [pltpu.SemaphoreType.DMA] * 2,
    ],
)
def _gin_aggregate(x_hbm, bond_hbm, src_hbm, dst_hbm, out_hbm,
                   si, di, rows, bondb, agg_sh, gsem, bsem):
    cid = lax.axis_index("c")
    sid = lax.axis_index("s")

    def _zero_row(i, _):
        for j in range(_LANES):
            rows[0][i, pl.ds(j * 16, 16)] = jnp.zeros((16,), jnp.float32)
        return 0
    lax.fori_loop(0, _CH, _zero_row, 0)
    nbase = sid * _RPS
    for k in range(_RPS // _CH):
        pltpu.sync_copy(rows[0], agg_sh.at[pl.ds(nbase + k * _CH, _CH)])
    plsc.subcore_barrier()

    ebase = (cid * _NS + sid) * _EPW

    def _issue(c, b):
        off = ebase + c * _CH
        pltpu.sync_copy(src_hbm.at[pl.ds(off, _CH)], si[b])
        pltpu.sync_copy(dst_hbm.at[pl.ds(off, _CH)], di[b])
        pltpu.async_copy(bond_hbm.at[pl.ds(off, _CH)], bondb[b], bsem[b])
        pltpu.async_copy(x_hbm.at[si[b]], rows[b], gsem[b])

    def _finish(b):
        pltpu.make_async_copy(
            bond_hbm.at[pl.ds(0, _CH)], bondb[b], bsem[b]).wait()
        pltpu.make_async_copy(x_hbm.at[si[b]], rows[b], gsem[b]).wait()

        @plsc.parallel_loop(0, _CH, unroll=8)
        def _row(r):
            for j in range(_LANES):
                sl = pl.ds(j * 16, 16)
                rows[b][r, sl] = jnp.maximum(
                    rows[b][r, sl] + bondb[b][r, sl], 0.0)
        pltpu.sync_copy(rows[b], agg_sh.at[di[b]], add=True)

    _issue(0, 0)

    def _pair(g, _):
        c0 = 2 * g
        _issue(c0 + 1, 1)
        _finish(0)
        _issue(c0 + 2, 0)
        _finish(1)
        return 0
    lax.fori_loop(0, _NCHUNK // 2 - 1, _pair, 0)
    _issue(_NCHUNK - 1, 1)
    _finish(0)
    _finish(1)
    plsc.subcore_barrier()

    obase = cid * _NPAD + nbase
    for k in range(_RPS // _CH):
        pltpu.sync_copy(agg_sh.at[pl.ds(nbase + k * _CH, _CH)], rows[0])
        pltpu.sync_copy(rows[0], out_hbm.at[pl.ds(obase + k * _CH, _CH)])


@functools.partial(
    pl.kernel,
    out_type=jax.ShapeDtypeStruct((_EP, _H), jnp.float32),
    mesh=_sc_mesh,
    scratch_types=[
        [pltpu.VMEM((_CHP,), jnp.int32)] * 2,
        [pltpu.VMEM((_CHP,), jnp.int32)] * 2,
        [pltpu.VMEM((_CHP, _H), jnp.float32)] * 2,
        [pltpu.VMEM((_CHP, _H), jnp.float32)] * 2,
        [pltpu.SemaphoreType.DMA] * 2,
        [pltpu.SemaphoreType.DMA] * 2,
    ],
)
def _edge_pair_product(x_hbm, src_hbm, dst_hbm, out_hbm,
                       si, di, av, bv, sem1, sem2):
    cid = lax.axis_index("c")
    sid = lax.axis_index("s")
    ebase = (cid * _NS + sid) * _EPW

    def _issue(c, b):
        off = ebase + c * _CHP
        pltpu.sync_copy(src_hbm.at[pl.ds(off, _CHP)], si[b])
        pltpu.sync_copy(dst_hbm.at[pl.ds(off, _CHP)], di[b])
        pltpu.async_copy(x_hbm.at[si[b]], av[b], sem1[b])
        pltpu.async_copy(x_hbm.at[di[b]], bv[b], sem2[b])

    def _finish(c, b):
        off = ebase + c * _CHP
        pltpu.make_async_copy(x_hbm.at[si[b]], av[b], sem1[b]).wait()
        pltpu.make_async_copy(x_hbm.at[di[b]], bv[b], sem2[b]).wait()

        @plsc.parallel_loop(0, _CHP, unroll=8)
        def _row(r):
            for j in range(_LANES):
                sl = pl.ds(j * 16, 16)
                av[b][r, sl] = av[b][r, sl] * bv[b][r, sl]
        pltpu.sync_copy(av[b], out_hbm.at[pl.ds(off, _CHP)])

    _issue(0, 0)

    def _pair(g, _):
        c0 = 2 * g
        _issue(c0 + 1, 1)
        _finish(c0, 0)
        _issue(c0 + 2, 0)
        _finish(c0 + 1, 1)
        return 0
    lax.fori_loop(0, _NCHUNKP // 2 - 1, _pair, 0)
    _issue(_NCHUNKP - 1, 1)
    _finish(_NCHUNKP - 2, 0)
    _finish(_NCHUNKP - 1, 1)


@functools.partial(
    pl.kernel,
    out_type=(jax.ShapeDtypeStruct((_EP,), jnp.float32),
              jax.ShapeDtypeStruct((_EP,), jnp.float32)),
    mesh=_sc_mesh,
    scratch_types=[
        pltpu.VMEM((_N,), jnp.int32),
        pltpu.VMEM((64,), jnp.float32),
        pltpu.VMEM((_NGRAPH,), jnp.int32),
        pltpu.VMEM((_NGRAPH,), jnp.float32),
        pltpu.VMEM((_NGRAPH,), jnp.float32),
        pltpu.VMEM((_EPW,), jnp.int32),
        pltpu.VMEM((_EPW,), jnp.float32),
        pltpu.VMEM((_EPW,), jnp.float32),
        pltpu.VMEM((_EPW,), jnp.float32),
    ],
    compiler_params=pltpu.CompilerParams(needs_layout_passes=False),
)
def _edge_sigma_target(batch_hbm, sig_hbm, nl_hbm, src_hbm, dn_hbm,
                       sig_out_hbm, tgt_out_hbm,
                       bt_v, sg_v, nl_v, us_v, ni_v, src_v, dn_v, so_v, to_v):
    cid = lax.axis_index("c")
    sid = lax.axis_index("s")
    pltpu.sync_copy(batch_hbm, bt_v)
    pltpu.sync_copy(sig_hbm, sg_v)
    pltpu.sync_copy(nl_hbm, nl_v)

    def _us(i, _):
        sl = pl.ds(i * 16, 16)
        sg = plsc.load_gather(sg_v, [nl_v[sl]])
        us_v[sl] = sg
        ni_v[sl] = -1.0 / (sg * sg)
        return 0
    lax.fori_loop(0, _NGRAPH // 16, _us, 0)

    ebase = (cid * _NS + sid) * _EPW
    pltpu.sync_copy(src_hbm.at[pl.ds(ebase, _EPW)], src_v)
    pltpu.sync_copy(dn_hbm.at[pl.ds(ebase, _EPW)], dn_v)

    @plsc.parallel_loop(0, _EPW // 16, unroll=4)
    def _e(i):
        sl = pl.ds(i * 16, 16)
        g = plsc.load_gather(bt_v, [src_v[sl]])
        so_v[sl] = plsc.load_gather(us_v, [g])
        to_v[sl] = plsc.load_gather(ni_v, [g]) * dn_v[sl]
    pltpu.sync_copy(so_v, sig_out_hbm.at[pl.ds(ebase, _EPW)])
    pltpu.sync_copy(to_v, tgt_out_hbm.at[pl.ds(ebase, _EPW)])


_BER = 32
_BE = _BER * _H
_BN = 2000


def _bond_tc(pd2, et2, Wi1, bi1, Wi2, bi2, emb_pad):
    def body(pd_ref, et_ref, w1_ref, b1_ref, w2_ref, b2_ref, emb_ref,
             out_ref):
        pdT = jnp.transpose(pd_ref[...])
        etT = jnp.transpose(et_ref[...])
        iota = lax.broadcasted_iota(jnp.int32, (_H, _H), 1).astype(jnp.float32)
        a_cols = []
        oh_cols = []
        for r in range(_BER):
            pcol = pdT[:, r:r + 1]
            a_cols.append(jnp.maximum(pcol * w1_ref[...] + b1_ref[...], 0.0))
            oh_cols.append((etT[:, r:r + 1] == iota).astype(jnp.float32))
        a = jnp.concatenate(a_cols, 0)
        oh = jnp.concatenate(oh_cols, 0)
        demb = jnp.dot(a, w2_ref[...],
                       preferred_element_type=jnp.float32) + b2_ref[...]
        battr = jnp.dot(oh, emb_ref[...],
                        preferred_element_type=jnp.float32,
                        precision=lax.Precision.HIGHEST)
        out_ref[...] = demb * battr

    return pl.pallas_call(
        body,
        grid=(_EP // _BE,),
        in_specs=[
            pl.BlockSpec((_BER, _H), lambda i: (i, 0)),
            pl.BlockSpec((_BER, _H), lambda i: (i, 0)),
            pl.BlockSpec((1, _H), lambda i: (0, 0)),
            pl.BlockSpec((1, _H), lambda i: (0, 0)),
            pl.BlockSpec((_H, _H), lambda i: (0, 0)),
            pl.BlockSpec((1, _H), lambda i: (0, 0)),
            pl.BlockSpec((_H, _H), lambda i: (0, 0)),
        ],
        out_specs=pl.BlockSpec((_BE, _H), lambda i: (i, 0)),
        out_shape=jax.ShapeDtypeStruct((_EP, _H), jnp.float32),
    )(pd2, et2, Wi1, bi1, Wi2, bi2, emb_pad)


def _node_emb_tc(nt, emb_pad):
    def body(nt_ref, emb_ref, out_ref):
        onehot = (nt_ref[...] == lax.broadcasted_iota(
            jnp.int32, (_BN, _H), 1)).astype(jnp.float32)
        out_ref[...] = jnp.dot(onehot, emb_ref[...],
                               preferred_element_type=jnp.float32,
                               precision=lax.Precision.HIGHEST)

    return pl.pallas_call(
        body,
        grid=(_N // _BN,),
        in_specs=[
            pl.BlockSpec((_BN, 1), lambda i: (i, 0)),
            pl.BlockSpec((_H, _H), lambda i: (0, 0)),
        ],
        out_specs=pl.BlockSpec((_BN, _H), lambda i: (i, 0)),
        out_shape=jax.ShapeDtypeStruct((_N, _H), jnp.float32),
    )(nt, emb_pad)


def _node_mlp_tc(x, p0, p1, W1, b1, W2, b2):
    def body(x_ref, p0_ref, p1_ref, w1_ref, b1_ref, w2_ref, b2_ref, out_ref):
        h = x_ref[...] + p0_ref[...] + p1_ref[...]
        t = jnp.maximum(jnp.dot(h, w1_ref[...],
                                preferred_element_type=jnp.float32)
                        + b1_ref[...], 0.0)
        u = jnp.dot(t, w2_ref[...],
                    preferred_element_type=jnp.float32) + b2_ref[...]
        out_ref[...] = jnp.maximum(u, 0.0) + x_ref[...]

    return pl.pallas_call(
        body,
        grid=(_N // _BN,),
        in_specs=[
            pl.BlockSpec((_BN, _H), lambda i: (i, 0)),
            pl.BlockSpec((_BN, _H), lambda i: (i, 0)),
            pl.BlockSpec((_BN, _H), lambda i: (i, 0)),
            pl.BlockSpec((_H, _H), lambda i: (0, 0)),
            pl.BlockSpec((1, _H), lambda i: (0, 0)),
            pl.BlockSpec((_H, _H), lambda i: (0, 0)),
            pl.BlockSpec((1, _H), lambda i: (0, 0)),
        ],
        out_specs=pl.BlockSpec((_BN, _H), lambda i: (i, 0)),
        out_shape=jax.ShapeDtypeStruct((_N, _H), jnp.float32),
    )(x, p0, p1, W1, b1, W2, b2)


def _score_tc(prod, bond, sig2, Wo1a, Wo1b, bo1, Wo2p, bo2p, Wo3p, bo3):
    def body(p_ref, b_ref, s_ref, w1a_ref, w1b_ref, b1_ref, w2_ref, b2_ref,
             w3_ref, b3_ref, out_ref):
        s1 = jnp.maximum(
            jnp.dot(p_ref[...], w1a_ref[...],
                    preferred_element_type=jnp.float32)
            + jnp.dot(b_ref[...], w1b_ref[...],
                      preferred_element_type=jnp.float32)
            + b1_ref[...], 0.0)
        s2 = jnp.maximum(jnp.dot(s1, w2_ref[...],
                                 preferred_element_type=jnp.float32)
                         + b2_ref[...], 0.0)
        raw = jnp.dot(s2, w3_ref[...],
                      preferred_element_type=jnp.float32) + b3_ref[...]
        sgT = jnp.transpose(s_ref[...])
        cols = []
        for r in range(_BER):
            rcol = raw[r * _H:(r + 1) * _H, 0:1]
            cols.append(rcol * (1.0 / sgT[:, r:r + 1]))
        outT = jnp.concatenate(cols, 1)
        out_ref[...] = jnp.transpose(outT)

    return pl.pallas_call(
        body,
        grid=(_EP // _BE,),
        in_specs=[
            pl.BlockSpec((_BE, _H), lambda i: (i, 0)),
            pl.BlockSpec((_BE, _H), lambda i: (i, 0)),
            pl.BlockSpec((_BER, _H), lambda i: (i, 0)),
            pl.BlockSpec((_H, _H), lambda i: (0, 0)),
            pl.BlockSpec((_H, _H), lambda i: (0, 0)),
            pl.BlockSpec((1, _H), lambda i: (0, 0)),
            pl.BlockSpec((_H, _H), lambda i: (0, 0)),
            pl.BlockSpec((1, _H), lambda i: (0, 0)),
            pl.BlockSpec((_H, 1), lambda i: (0, 0)),
            pl.BlockSpec((1, 1), lambda i: (0, 0)),
        ],
        out_specs=pl.BlockSpec((_BER, _H), lambda i: (i, 0)),
        out_shape=jax.ShapeDtypeStruct((_ER, _H), jnp.float32),
    )(prod, bond, sig2, Wo1a, Wo1b, bo1, Wo2p, bo2p, Wo3p, bo3)


def kernel(node_type, edge_type, edge_index, batch, edge_length,
           node_emb, edge_emb, Wi1, bi1, Wi2, bi2,
           convW1, convb1, convW2, convb2,
           Wo1, bo1, Wo2, bo2, Wo3, bo3):
    f32 = jnp.float32
    i32 = jnp.int32
    sigmas = jnp.exp(
        jnp.linspace(jnp.log(10.0), jnp.log(0.01), _NLEV)).astype(f32)
    kn = jax.random.key(42)
    noise_level = jax.random.randint(
        jax.random.fold_in(kn, 0), (_NGRAPH,), 0, _NLEV)
    d_noise = jax.random.normal(
        jax.random.fold_in(kn, 1), (_E,), dtype=f32)

    npad = _EP - _E
    fill = jnp.arange(npad, dtype=i32)
    src = jnp.concatenate([edge_index[0].astype(i32), fill % _N])
    dst = edge_index[1].astype(i32)
    dst_gin = jnp.concatenate([dst, _N + (fill % (_NPAD - _N))])
    dst_pair = jnp.concatenate([dst, fill % _N])
    sig_pad = jnp.pad(sigmas, (0, 64 - _NLEV))
    node_emb_pad = jnp.pad(node_emb, ((0, _H - node_emb.shape[0]), (0, 0)))
    edge_emb_pad = jnp.pad(edge_emb, ((0, _H - edge_emb.shape[0]), (0, 0)))

    pd2 = jnp.pad(edge_length[:, 0] + d_noise, (0, npad)).reshape(_ER, _H)
    et2 = jnp.pad(edge_type.astype(f32), (0, npad)).reshape(_ER, _H)
    dn_pad = jnp.pad(d_noise, (0, npad))

    bond = _bond_tc(pd2, et2, Wi1, bi1[None], Wi2, bi2[None], edge_emb_pad)
    x = _node_emb_tc(node_type.astype(i32)[:, None], node_emb_pad)
    for i in range(_NCONV):
        part = _gin_aggregate(x, bond, src, dst_gin)
        x = _node_mlp_tc(x, part[:_N], part[_NPAD:_NPAD + _N],
                         convW1[i], convb1[i][None],
                         convW2[i], convb2[i][None])

    prod = _edge_pair_product(x, src, dst_pair)
    sig_flat, tgt_flat = _edge_sigma_target(
        batch.astype(i32), sig_pad, noise_level.astype(i32), src, dn_pad)

    Wo2p = jnp.pad(Wo2, ((0, 0), (0, _H - Wo2.shape[1])))
    bo2p = jnp.pad(bo2, (0, _H - bo2.shape[0]))
    Wo3p = jnp.pad(Wo3, ((0, _H - Wo3.shape[0]), (0, 0)))
    scores2 = _score_tc(prod, bond, sig_flat.reshape(_ER, _H),
                        Wo1[:_H], Wo1[_H:], bo1[None],
                        Wo2p, bo2p[None], Wo3p, bo3[None])
    scores = scores2.reshape(_EP)[:_E, None]
    return (scores, tgt_flat[:_E, None], sig_flat[:_E, None])


# speedup vs baseline: 2.2328x; 1.1894x over previous
import functools

import jax
import jax.numpy as jnp
from jax import lax
from jax.experimental import pallas as pl
from jax.experimental.pallas import tpu as pltpu
from jax.experimental.pallas import tpu_sc as plsc

_N = 10000
_E = 320000
_H = 128
_NCONV = 4
_NGRAPH = 256
_NLEV = 50

_EP = 327680
_ER = _EP // _H
_NC = 2
_NS = 16
_NW = _NC * _NS
_EPW = _EP // _NW
_CH = 64
_NCHUNK = _EPW // _CH
_CHP = 128
_NCHUNKP = _EPW // _CHP
_NPAD = 10240
_RPS = _NPAD // _NS
_RZ = 128
_LANES = _H // 16

_sc_mesh = plsc.VectorSubcoreMesh(
    core_axis_name="c", subcore_axis_name="s", num_cores=_NC, num_subcores=_NS)


@functools.partial(
    pl.kernel,
    out_type=jax.ShapeDtypeStruct((_NC * _NPAD, _H), jnp.float32),
    mesh=_sc_mesh,
    scratch_types=[
        pltpu.VMEM((_EPW,), jnp.int32),
        [pltpu.VMEM((_CH,), jnp.int32)] * 2,
        [pltpu.VMEM((_CH, _H), jnp.float32)] * 2,
        [pltpu.VMEM((_CH, _H), jnp.float32)] * 2,
        pltpu.VMEM_SHARED((_NPAD, _H), jnp.float32),
        [pltpu.SemaphoreType.DMA] * 2,
        [pltpu.SemaphoreType.DMA] * 2,
        [pltpu.SemaphoreType.DMA] * 2,
    ],
)
def _gin_aggregate(x_hbm, bond_hbm, src_hbm, dst_hbm, out_hbm,
                   srca, di, rows, bondb, agg_sh, gsem, bsem, dsem):
    cid = lax.axis_index("c")
    sid = lax.axis_index("s")

    def _zero_row(i, _):
        for j in range(_LANES):
            rows[0][i, pl.ds(j * 16, 16)] = jnp.zeros((16,), jnp.float32)
        return 0
    lax.fori_loop(0, _CH, _zero_row, 0)
    nbase = sid * _RPS
    for k in range(_RPS // _CH):
        pltpu.sync_copy(rows[0], agg_sh.at[pl.ds(nbase + k * _CH, _CH)])

    ebase = (cid * _NS + sid) * _EPW
    pltpu.sync_copy(src_hbm.at[pl.ds(ebase, _EPW)], srca)
    plsc.subcore_barrier()

    def _issue(c, b):
        off = ebase + c * _CH
        pltpu.async_copy(dst_hbm.at[pl.ds(off, _CH)], di[b], dsem[b])
        pltpu.async_copy(bond_hbm.at[pl.ds(off, _CH)], bondb[b], bsem[b])
        pltpu.async_copy(
            x_hbm.at[srca.at[pl.ds(c * _CH, _CH)]], rows[b], gsem[b])

    def _finish(c, b):
        pltpu.make_async_copy(
            bond_hbm.at[pl.ds(0, _CH)], bondb[b], bsem[b]).wait()
        pltpu.make_async_copy(
            x_hbm.at[srca.at[pl.ds(c * _CH, _CH)]], rows[b], gsem[b]).wait()

        @plsc.parallel_loop(0, _CH, unroll=8)
        def _row(r):
            for j in range(_LANES):
                sl = pl.ds(j * 16, 16)
                rows[b][r, sl] = jnp.maximum(
                    rows[b][r, sl] + bondb[b][r, sl], 0.0)
        pltpu.make_async_copy(
            dst_hbm.at[pl.ds(0, _CH)], di[b], dsem[b]).wait()
        pltpu.sync_copy(rows[b], agg_sh.at[di[b]], add=True)

    _issue(0, 0)

    def _pair(g, _):
        c0 = 2 * g
        _issue(c0 + 1, 1)
        _finish(c0, 0)
        _issue(c0 + 2, 0)
        _finish(c0 + 1, 1)
        return 0
    lax.fori_loop(0, _NCHUNK // 2 - 1, _pair, 0)
    _issue(_NCHUNK - 1, 1)
    _finish(_NCHUNK - 2, 0)
    _finish(_NCHUNK - 1, 1)
    plsc.subcore_barrier()

    obase = cid * _NPAD + nbase
    for k in range(_RPS // _CH):
        pltpu.sync_copy(agg_sh.at[pl.ds(nbase + k * _CH, _CH)], rows[0])
        pltpu.sync_copy(rows[0], out_hbm.at[pl.ds(obase + k * _CH, _CH)])


@functools.partial(
    pl.kernel,
    out_type=jax.ShapeDtypeStruct((_EP, _H), jnp.float32),
    mesh=_sc_mesh,
    scratch_types=[
        pltpu.VMEM((_EPW,), jnp.int32),
        pltpu.VMEM((_EPW,), jnp.int32),
        [pltpu.VMEM((_CHP, _H), jnp.float32)] * 2,
        [pltpu.VMEM((_CHP, _H), jnp.float32)] * 2,
        [pltpu.SemaphoreType.DMA] * 2,
        [pltpu.SemaphoreType.DMA] * 2,
    ],
)
def _edge_pair_product(x_hbm, src_hbm, dst_hbm, out_hbm,
                       srca, dsta, av, bv, sem1, sem2):
    cid = lax.axis_index("c")
    sid = lax.axis_index("s")
    ebase = (cid * _NS + sid) * _EPW
    pltpu.sync_copy(src_hbm.at[pl.ds(ebase, _EPW)], srca)
    pltpu.sync_copy(dst_hbm.at[pl.ds(ebase, _EPW)], dsta)

    def _issue(c, b):
        lo = c * _CHP
        pltpu.async_copy(
            x_hbm.at[srca.at[pl.ds(lo, _CHP)]], av[b], sem1[b])
        pltpu.async_copy(
            x_hbm.at[dsta.at[pl.ds(lo, _CHP)]], bv[b], sem2[b])

    def _finish(c, b):
        lo = c * _CHP
        pltpu.make_async_copy(
            x_hbm.at[srca.at[pl.ds(lo, _CHP)]], av[b], sem1[b]).wait()
        pltpu.make_async_copy(
            x_hbm.at[dsta.at[pl.ds(lo, _CHP)]], bv[b], sem2[b]).wait()

        @plsc.parallel_loop(0, _CHP, unroll=8)
        def _row(r):
            for j in range(_LANES):
                sl = pl.ds(j * 16, 16)
                av[b][r, sl] = av[b][r, sl] * bv[b][r, sl]
        pltpu.sync_copy(av[b], out_hbm.at[pl.ds(ebase + lo, _CHP)])

    _issue(0, 0)

    def _pair(g, _):
        c0 = 2 * g
        _issue(c0 + 1, 1)
        _finish(c0, 0)
        _issue(c0 + 2, 0)
        _finish(c0 + 1, 1)
        return 0
    lax.fori_loop(0, _NCHUNKP // 2 - 1, _pair, 0)
    _issue(_NCHUNKP - 1, 1)
    _finish(_NCHUNKP - 2, 0)
    _finish(_NCHUNKP - 1, 1)


@functools.partial(
    pl.kernel,
    out_type=(jax.ShapeDtypeStruct((_EP,), jnp.float32),
              jax.ShapeDtypeStruct((_EP,), jnp.float32)),
    mesh=_sc_mesh,
    scratch_types=[
        pltpu.VMEM((_N,), jnp.int32),
        pltpu.VMEM((64,), jnp.float32),
        pltpu.VMEM((_NGRAPH,), jnp.int32),
        pltpu.VMEM((_NGRAPH,), jnp.float32),
        pltpu.VMEM((_NGRAPH,), jnp.float32),
        pltpu.VMEM((_EPW,), jnp.int32),
        pltpu.VMEM((_EPW,), jnp.float32),
        pltpu.VMEM((_EPW,), jnp.float32),
        pltpu.VMEM((_EPW,), jnp.float32),
    ],
    compiler_params=pltpu.CompilerParams(needs_layout_passes=False),
)
def _edge_sigma_target(batch_hbm, sig_hbm, nl_hbm, src_hbm, dn_hbm,
                       sig_out_hbm, tgt_out_hbm,
                       bt_v, sg_v, nl_v, us_v, ni_v, src_v, dn_v, so_v, to_v):
    cid = lax.axis_index("c")
    sid = lax.axis_index("s")
    pltpu.sync_copy(batch_hbm, bt_v)
    pltpu.sync_copy(sig_hbm, sg_v)
    pltpu.sync_copy(nl_hbm, nl_v)

    def _us(i, _):
        sl = pl.ds(i * 16, 16)
        sg = plsc.load_gather(sg_v, [nl_v[sl]])
        us_v[sl] = sg
        ni_v[sl] = -1.0 / (sg * sg)
        return 0
    lax.fori_loop(0, _NGRAPH // 16, _us, 0)

    ebase = (cid * _NS + sid) * _EPW
    pltpu.sync_copy(src_hbm.at[pl.ds(ebase, _EPW)], src_v)
    pltpu.sync_copy(dn_hbm.at[pl.ds(ebase, _EPW)], dn_v)

    @plsc.parallel_loop(0, _EPW // 16, unroll=4)
    def _e(i):
        sl = pl.ds(i * 16, 16)
        g = plsc.load_gather(bt_v, [src_v[sl]])
        so_v[sl] = plsc.load_gather(us_v, [g])
        to_v[sl] = plsc.load_gather(ni_v, [g]) * dn_v[sl]
    pltpu.sync_copy(so_v, sig_out_hbm.at[pl.ds(ebase, _EPW)])
    pltpu.sync_copy(to_v, tgt_out_hbm.at[pl.ds(ebase, _EPW)])


_BER = 32
_BE = _BER * _H
_BN = 2000


def _bond_tc(pd2, et2, Wi1, bi1, Wi2, bi2, emb_pad):
    def body(pd_ref, et_ref, w1_ref, b1_ref, w2_ref, b2_ref, emb_ref,
             out_ref):
        pdT = jnp.transpose(pd_ref[...])
        etT = jnp.transpose(et_ref[...])
        iota = lax.broadcasted_iota(jnp.int32, (_H, _H), 1).astype(jnp.float32)
        a_cols = []
        oh_cols = []
        for r in range(_BER):
            pcol = pdT[:, r:r + 1]
            a_cols.append(jnp.maximum(pcol * w1_ref[...] + b1_ref[...], 0.0))
            oh_cols.append((etT[:, r:r + 1] == iota).astype(jnp.float32))
        a = jnp.concatenate(a_cols, 0)
        oh = jnp.concatenate(oh_cols, 0)
        demb = jnp.dot(a, w2_ref[...],
                       preferred_element_type=jnp.float32) + b2_ref[...]
        battr = jnp.dot(oh, emb_ref[...],
                        preferred_element_type=jnp.float32,
                        precision=lax.Precision.HIGHEST)
        out_ref[...] = demb * battr

    return pl.pallas_call(
        body,
        grid=(_EP // _BE,),
        in_specs=[
            pl.BlockSpec((_BER, _H), lambda i: (i, 0)),
            pl.BlockSpec((_BER, _H), lambda i: (i, 0)),
            pl.BlockSpec((1, _H), lambda i: (0, 0)),
            pl.BlockSpec((1, _H), lambda i: (0, 0)),
            pl.BlockSpec((_H, _H), lambda i: (0, 0)),
            pl.BlockSpec((1, _H), lambda i: (0, 0)),
            pl.BlockSpec((_H, _H), lambda i: (0, 0)),
        ],
        out_specs=pl.BlockSpec((_BE, _H), lambda i: (i, 0)),
        out_shape=jax.ShapeDtypeStruct((_EP, _H), jnp.float32),
    )(pd2, et2, Wi1, bi1, Wi2, bi2, emb_pad)


def _node_emb_tc(nt, emb_pad):
    def body(nt_ref, emb_ref, out_ref):
        onehot = (nt_ref[...] == lax.broadcasted_iota(
            jnp.int32, (_BN, _H), 1)).astype(jnp.float32)
        out_ref[...] = jnp.dot(onehot, emb_ref[...],
                               preferred_element_type=jnp.float32,
                               precision=lax.Precision.HIGHEST)

    return pl.pallas_call(
        body,
        grid=(_N // _BN,),
        in_specs=[
            pl.BlockSpec((_BN, 1), lambda i: (i, 0)),
            pl.BlockSpec((_H, _H), lambda i: (0, 0)),
        ],
        out_specs=pl.BlockSpec((_BN, _H), lambda i: (i, 0)),
        out_shape=jax.ShapeDtypeStruct((_N, _H), jnp.float32),
    )(nt, emb_pad)


def _node_mlp_tc(x, p0, p1, W1, b1, W2, b2):
    def body(x_ref, p0_ref, p1_ref, w1_ref, b1_ref, w2_ref, b2_ref, out_ref):
        h = x_ref[...] + p0_ref[...] + p1_ref[...]
        t = jnp.maximum(jnp.dot(h, w1_ref[...],
                                preferred_element_type=jnp.float32)
                        + b1_ref[...], 0.0)
        u = jnp.dot(t, w2_ref[...],
                    preferred_element_type=jnp.float32) + b2_ref[...]
        out_ref[...] = jnp.maximum(u, 0.0) + x_ref[...]

    return pl.pallas_call(
        body,
        grid=(_N // _BN,),
        in_specs=[
            pl.BlockSpec((_BN, _H), lambda i: (i, 0)),
            pl.BlockSpec((_BN, _H), lambda i: (i, 0)),
            pl.BlockSpec((_BN, _H), lambda i: (i, 0)),
            pl.BlockSpec((_H, _H), lambda i: (0, 0)),
            pl.BlockSpec((1, _H), lambda i: (0, 0)),
            pl.BlockSpec((_H, _H), lambda i: (0, 0)),
            pl.BlockSpec((1, _H), lambda i: (0, 0)),
        ],
        out_specs=pl.BlockSpec((_BN, _H), lambda i: (i, 0)),
        out_shape=jax.ShapeDtypeStruct((_N, _H), jnp.float32),
    )(x, p0, p1, W1, b1, W2, b2)


def _score_tc(prod, bond, sig2, Wo1a, Wo1b, bo1, Wo2p, bo2p, Wo3p, bo3):
    def body(p_ref, b_ref, s_ref, w1a_ref, w1b_ref, b1_ref, w2_ref, b2_ref,
             w3_ref, b3_ref, out_ref):
        s1 = jnp.maximum(
            jnp.dot(p_ref[...], w1a_ref[...],
                    preferred_element_type=jnp.float32)
            + jnp.dot(b_ref[...], w1b_ref[...],
                      preferred_element_type=jnp.float32)
            + b1_ref[...], 0.0)
        s2 = jnp.maximum(jnp.dot(s1, w2_ref[...],
                                 preferred_element_type=jnp.float32)
                         + b2_ref[...], 0.0)
        raw = jnp.dot(s2, w3_ref[...],
                      preferred_element_type=jnp.float32) + b3_ref[...]
        sgT = jnp.transpose(s_ref[...])
        cols = []
        for r in range(_BER):
            rcol = raw[r * _H:(r + 1) * _H, 0:1]
            cols.append(rcol * (1.0 / sgT[:, r:r + 1]))
        outT = jnp.concatenate(cols, 1)
        out_ref[...] = jnp.transpose(outT)

    return pl.pallas_call(
        body,
        grid=(_EP // _BE,),
        in_specs=[
            pl.BlockSpec((_BE, _H), lambda i: (i, 0)),
            pl.BlockSpec((_BE, _H), lambda i: (i, 0)),
            pl.BlockSpec((_BER, _H), lambda i: (i, 0)),
            pl.BlockSpec((_H, _H), lambda i: (0, 0)),
            pl.BlockSpec((_H, _H), lambda i: (0, 0)),
            pl.BlockSpec((1, _H), lambda i: (0, 0)),
            pl.BlockSpec((_H, _H), lambda i: (0, 0)),
            pl.BlockSpec((1, _H), lambda i: (0, 0)),
            pl.BlockSpec((_H, 1), lambda i: (0, 0)),
            pl.BlockSpec((1, 1), lambda i: (0, 0)),
        ],
        out_specs=pl.BlockSpec((_BER, _H), lambda i: (i, 0)),
        out_shape=jax.ShapeDtypeStruct((_ER, _H), jnp.float32),
    )(prod, bond, sig2, Wo1a, Wo1b, bo1, Wo2p, bo2p, Wo3p, bo3)


def kernel(node_type, edge_type, edge_index, batch, edge_length,
           node_emb, edge_emb, Wi1, bi1, Wi2, bi2,
           convW1, convb1, convW2, convb2,
           Wo1, bo1, Wo2, bo2, Wo3, bo3):
    f32 = jnp.float32
    i32 = jnp.int32
    sigmas = jnp.exp(
        jnp.linspace(jnp.log(10.0), jnp.log(0.01), _NLEV)).astype(f32)
    kn = jax.random.key(42)
    noise_level = jax.random.randint(
        jax.random.fold_in(kn, 0), (_NGRAPH,), 0, _NLEV)
    d_noise = jax.random.normal(
        jax.random.fold_in(kn, 1), (_E,), dtype=f32)

    npad = _EP - _E
    fill = jnp.arange(npad, dtype=i32)
    src = jnp.concatenate([edge_index[0].astype(i32), fill % _N])
    dst = edge_index[1].astype(i32)
    dst_gin = jnp.concatenate([dst, _N + (fill % (_NPAD - _N))])
    dst_pair = jnp.concatenate([dst, fill % _N])
    sig_pad = jnp.pad(sigmas, (0, 64 - _NLEV))
    node_emb_pad = jnp.pad(node_emb, ((0, _H - node_emb.shape[0]), (0, 0)))
    edge_emb_pad = jnp.pad(edge_emb, ((0, _H - edge_emb.shape[0]), (0, 0)))

    pd2 = jnp.pad(edge_length[:, 0] + d_noise, (0, npad)).reshape(_ER, _H)
    et2 = jnp.pad(edge_type.astype(f32), (0, npad)).reshape(_ER, _H)
    dn_pad = jnp.pad(d_noise, (0, npad))

    bond = _bond_tc(pd2, et2, Wi1, bi1[None], Wi2, bi2[None], edge_emb_pad)
    x = _node_emb_tc(node_type.astype(i32)[:, None], node_emb_pad)
    for i in range(_NCONV):
        part = _gin_aggregate(x, bond, src, dst_gin)
        x = _node_mlp_tc(x, part[:_N], part[_NPAD:_NPAD + _N],
                         convW1[i], convb1[i][None],
                         convW2[i], convb2[i][None])

    prod = _edge_pair_product(x, src, dst_pair)
    sig_flat, tgt_flat = _edge_sigma_target(
        batch.astype(i32), sig_pad, noise_level.astype(i32), src, dn_pad)

    Wo2p = jnp.pad(Wo2, ((0, 0), (0, _H - Wo2.shape[1])))
    bo2p = jnp.pad(bo2, (0, _H - bo2.shape[0]))
    Wo3p = jnp.pad(Wo3, ((0, _H - Wo3.shape[0]), (0, 0)))
    scores2 = _score_tc(prod, bond, sig_flat.reshape(_ER, _H),
                        Wo1[:_H], Wo1[_H:], bo1[None],
                        Wo2p, bo2p[None], Wo3p, bo3[None])
    scores = scores2.reshape(_EP)[:_E, None]
    return (scores, tgt_flat[:_E, None], sig_flat[:_E, None])
